# baseline (speedup 1.0000x reference)
import functools

import jax
import jax.numpy as jnp
from jax import lax
from jax.experimental import pallas as pl
from jax.experimental.pallas import tpu as pltpu
from jax.experimental.pallas import tpu_sc as plsc

B = 16384
D = 256
L = 16
NC = 2
NS = 16
NW = NC * NS
BPW = B // NW

_mesh = plsc.VectorSubcoreMesh(core_axis_name="c", subcore_axis_name="s")

_GDN = lax.GatherDimensionNumbers(
    offset_dims=(), collapsed_slice_dims=(0,), start_index_map=(0,))


def _lanes(x, lane_idx):
    idx = (lane_idx & (L - 1)).reshape(L, 1)
    return lax.gather(x, idx, dimension_numbers=_GDN, slice_sizes=(1,),
                      mode=lax.GatherScatterMode.PROMISE_IN_BOUNDS)


@functools.partial(
    pl.kernel,
    mesh=_mesh,
    out_type=jax.ShapeDtypeStruct((B, D), jnp.float32),
    scratch_types=[
        pltpu.VMEM((BPW,), jnp.int32),
        pltpu.VMEM((8, D), jnp.float32),
        pltpu.SemaphoreType.DMA,
        pltpu.SemaphoreType.DMA,
    ],
)
def _lookup(idx_hbm, table_hbm, out_hbm, idx_v, pat_v, sem, bsem):
    wid = lax.axis_index("s") * NC + lax.axis_index("c")
    base = wid * BPW
    pltpu.sync_copy(idx_hbm.at[wid], idx_v)
    builds = []
    for p in range(4):
        for h in range(2):
            bit = (p >> (1 - h)) & 1
            builds.append(pltpu.async_copy(
                table_hbm.at[pl.ds(bit, 1)],
                pat_v.at[pl.ds(2 * p + h, 1)], bsem))
    for cp in builds:
        cp.wait()

    two_iota = lax.iota(jnp.int32, L) * 2
    evens = two_iota
    odds = two_iota + 1

    def grp(g, carry):
        iv = idx_v[pl.ds(g * L, L)]
        pv = _lanes(iv, evens) * 2 + _lanes(iv, odds)
        for j in range(L // 2):
            p = pv[j]
            pltpu.async_copy(
                pat_v.at[pl.ds(2 * p, 2)],
                out_hbm.at[pl.ds(base + g * L + 2 * j, 2)],
                sem)
        return carry

    lax.fori_loop(0, BPW // L, grp, 0)
    my_out = out_hbm.at[pl.ds(base, BPW)]
    pltpu.make_async_copy(my_out, my_out, sem).wait()


def kernel(harmful, emb_weight):
    idx = jnp.reshape(harmful.astype(jnp.int32), (NW, BPW))
    return _lookup(idx, emb_weight)
